# baseline (speedup 1.0000x reference)
import functools

import jax
import jax.numpy as jnp
from jax import lax
from jax.experimental import pallas as pl
from jax.experimental.pallas import tpu as pltpu
from jax.experimental.pallas import tpu_sc as plsc

D_MODEL = 2048
NUM_EXPERTS = 16
NUM_SELECTED = 2
CAPACITY_FACTOR = 1.25
Z_LOSS_COEF = 0.01

TOKEN_BLOCK = 512
NEG_HUGE = -3.0e38

N_TOKENS = 16384
NW = 32
TOK_PER_W = N_TOKENS // NW
LANES = 16
GROUPS = TOK_PER_W // LANES


def _gate_block(w_ref, x_ref, lt_ref, lse_ref, ent_ref):
    step = pl.program_id(0)

    logits = lax.dot_general(
        w_ref[...], x_ref[...],
        dimension_numbers=(((1,), (1,)), ((), ())),
        preferred_element_type=jnp.float32)
    lt_ref[...] = logits

    m = jnp.max(logits, axis=0, keepdims=True)
    e = jnp.exp(logits - m)
    s = jnp.sum(e, axis=0, keepdims=True)
    lse = m + jnp.log(s)
    sel = jnp.sum(e * logits, axis=0, keepdims=True)
    ent = lse - sel / s
    block_lse = jnp.sum(lse)[None, None]
    block_ent = jnp.sum(ent)[None, None]

    @pl.when(step == 0)
    def _init():
        lse_ref[...] = block_lse
        ent_ref[...] = block_ent

    @pl.when(step != 0)
    def _acc():
        lse_ref[...] += block_lse
        ent_ref[...] += block_ent


def _route_sc_body(lt_hbm, oi_hbm, ow_hbm, oc_hbm,
                   lt_v, i1_v, i2_v, w1_v, w2_v, acc_v):
    wid = lax.axis_index("s") * 2 + lax.axis_index("c")
    base = wid * TOK_PER_W
    pltpu.sync_copy(lt_hbm.at[:, pl.ds(base, TOK_PER_W)], lt_v)

    zeros = jnp.zeros((LANES,), jnp.float32)
    for e in range(NUM_EXPERTS):
        acc_v[e, :] = zeros

    def body(g, _):
        off = pl.multiple_of(g * LANES, LANES)
        vs = [lt_v[e, pl.ds(off, LANES)] for e in range(NUM_EXPERTS)]

        m = vs[0]
        for e in range(1, NUM_EXPERTS):
            m = jnp.maximum(m, vs[e])

        i1 = jnp.full((LANES,), NUM_EXPERTS, jnp.int32)
        for e in range(NUM_EXPERTS - 1, -1, -1):
            i1 = jnp.where(vs[e] == m, jnp.full((LANES,), e, jnp.int32), i1)

        neg = jnp.full((LANES,), NEG_HUGE, jnp.float32)
        vm = [jnp.where(i1 == jnp.full((LANES,), e, jnp.int32), neg, vs[e])
              for e in range(NUM_EXPERTS)]
        l2 = vm[0]
        for e in range(1, NUM_EXPERTS):
            l2 = jnp.maximum(l2, vm[e])
        i2 = jnp.full((LANES,), NUM_EXPERTS, jnp.int32)
        for e in range(NUM_EXPERTS - 1, -1, -1):
            i2 = jnp.where(vm[e] == l2, jnp.full((LANES,), e, jnp.int32), i2)

        s = jnp.exp(vs[0] - m)
        for e in range(1, NUM_EXPERTS):
            s = s + jnp.exp(vs[e] - m)
        rs = 1.0 / s
        p1 = rs
        p2 = jnp.exp(l2 - m) * rs
        rden = 1.0 / (p1 + p2 + 1e-8)
        w1 = p1 * rden
        w2 = p2 * rden

        i1_v[pl.ds(off, LANES)] = i1
        i2_v[pl.ds(off, LANES)] = i2
        w1_v[pl.ds(off, LANES)] = w1
        w2_v[pl.ds(off, LANES)] = w2

        one = jnp.full((LANES,), 1.0, jnp.float32)
        for e in range(NUM_EXPERTS):
            ide = jnp.full((LANES,), e, jnp.int32)
            hit = jnp.where(i1 == ide, one, zeros) + \
                jnp.where(i2 == ide, one, zeros)
            acc_v[e, :] = acc_v[e, :] + hit
        return 0

    lax.fori_loop(0, GROUPS, body, 0)

    pltpu.sync_copy(i1_v, oi_hbm.at[0, pl.ds(base, TOK_PER_W)])
    pltpu.sync_copy(i2_v, oi_hbm.at[1, pl.ds(base, TOK_PER_W)])
    pltpu.sync_copy(w1_v, ow_hbm.at[0, pl.ds(base, TOK_PER_W)])
    pltpu.sync_copy(w2_v, ow_hbm.at[1, pl.ds(base, TOK_PER_W)])
    pltpu.sync_copy(acc_v, oc_hbm.at[wid])


@jax.jit
def _router(gate_weight, hidden_flat):
    n_tokens = hidden_flat.shape[0]
    grid = (n_tokens // TOKEN_BLOCK,)
    lt, lse_sum, ent_sum = pl.pallas_call(
        _gate_block,
        grid=grid,
        in_specs=[
            pl.BlockSpec((NUM_EXPERTS, D_MODEL), lambda i: (0, 0)),
            pl.BlockSpec((TOKEN_BLOCK, D_MODEL), lambda i: (i, 0)),
        ],
        out_specs=(
            pl.BlockSpec((NUM_EXPERTS, TOKEN_BLOCK), lambda i: (0, i)),
            pl.BlockSpec((1, 1), lambda i: (0, 0)),
            pl.BlockSpec((1, 1), lambda i: (0, 0)),
        ),
        out_shape=(
            jax.ShapeDtypeStruct((NUM_EXPERTS, n_tokens), jnp.float32),
            jax.ShapeDtypeStruct((1, 1), jnp.float32),
            jax.ShapeDtypeStruct((1, 1), jnp.float32),
        ),
        compiler_params=pltpu.CompilerParams(
            dimension_semantics=("arbitrary",),
        ),
    )(gate_weight, hidden_flat)

    mesh = plsc.VectorSubcoreMesh(core_axis_name="c", subcore_axis_name="s")
    route = functools.partial(
        pl.kernel, mesh=mesh,
        out_type=(
            jax.ShapeDtypeStruct((NUM_SELECTED, n_tokens), jnp.int32),
            jax.ShapeDtypeStruct((NUM_SELECTED, n_tokens), jnp.float32),
            jax.ShapeDtypeStruct((NW, NUM_EXPERTS, LANES), jnp.float32),
        ),
        scratch_types=[
            pltpu.VMEM((NUM_EXPERTS, TOK_PER_W), jnp.float32),
            pltpu.VMEM((TOK_PER_W,), jnp.int32),
            pltpu.VMEM((TOK_PER_W,), jnp.int32),
            pltpu.VMEM((TOK_PER_W,), jnp.float32),
            pltpu.VMEM((TOK_PER_W,), jnp.float32),
            pltpu.VMEM((NUM_EXPERTS, LANES), jnp.float32),
        ],
    )(_route_sc_body)
    it, wt, acc = route(lt)
    return it, wt, acc, lse_sum, ent_sum


def kernel(hidden_states, gate_weight):
    batch_size, seq_len, d_model = hidden_states.shape
    num_tokens = batch_size * seq_len
    hidden_flat = hidden_states.reshape(num_tokens, d_model)

    it, wt, acc, lse_sum, ent_sum = _router(gate_weight, hidden_flat)

    expert_counts = jnp.sum(acc, axis=(0, 2))
    capacity = int(CAPACITY_FACTOR * num_tokens / NUM_EXPERTS * NUM_SELECTED)
    expert_overflow = jnp.sum(jnp.maximum(expert_counts - capacity, 0.0))
    capacity_overflow_pct = expert_overflow / num_tokens * 100.0
    z_loss = lse_sum[0, 0] / num_tokens * Z_LOSS_COEF
    gate_entropy = ent_sum[0, 0] / num_tokens
    expert_load_normalized = expert_counts / jnp.sum(expert_counts)
    ideal_load = 1.0 / NUM_EXPERTS
    expert_load_variance = jnp.mean((expert_load_normalized - ideal_load) ** 2)

    expert_indices = it.T.reshape(batch_size, seq_len, NUM_SELECTED)
    expert_weights = wt.T.reshape(batch_size, seq_len, NUM_SELECTED)
    routing_confidence = wt[0]
    return (expert_indices, expert_weights, expert_counts,
            capacity_overflow_pct, z_loss, gate_entropy,
            expert_load_variance, routing_confidence)


# speedup vs baseline: 1.0422x; 1.0422x over previous
import functools

import jax
import jax.numpy as jnp
from jax import lax
from jax.experimental import pallas as pl
from jax.experimental.pallas import tpu as pltpu
from jax.experimental.pallas import tpu_sc as plsc

D_MODEL = 2048
NUM_EXPERTS = 16
NUM_SELECTED = 2
CAPACITY_FACTOR = 1.25
Z_LOSS_COEF = 0.01

TOKEN_BLOCK = 1024
NEG_HUGE = -3.0e38

N_TOKENS = 16384
N_CHUNKS = 2
CHUNK = N_TOKENS // N_CHUNKS
NW = 32
LANES = 16


def _gate_block(w_ref, x_ref, lt_ref, lse_ref, ent_ref):
    step = pl.program_id(0)

    logits = lax.dot_general(
        w_ref[...], x_ref[...],
        dimension_numbers=(((1,), (1,)), ((), ())),
        preferred_element_type=jnp.float32)
    lt_ref[...] = logits

    m = jnp.max(logits, axis=0, keepdims=True)
    e = jnp.exp(logits - m)
    s = jnp.sum(e, axis=0, keepdims=True)
    lse = m + jnp.log(s)
    sel = jnp.sum(e * logits, axis=0, keepdims=True)
    ent = lse - sel / s
    block_lse = jnp.sum(lse)[None, None]
    block_ent = jnp.sum(ent)[None, None]

    @pl.when(step == 0)
    def _init():
        lse_ref[...] = block_lse
        ent_ref[...] = block_ent

    @pl.when(step != 0)
    def _acc():
        lse_ref[...] += block_lse
        ent_ref[...] += block_ent


def _route_sc_body(lt_hbm, oi_hbm, ow_hbm, oc_hbm,
                   lt_v, i1_v, i2_v, w1_v, w2_v, acc_v,
                   *, tok_per_w):
    groups = tok_per_w // LANES
    wid = lax.axis_index("s") * 2 + lax.axis_index("c")
    base = wid * tok_per_w
    pltpu.sync_copy(lt_hbm.at[:, pl.ds(base, tok_per_w)], lt_v)

    zeros = jnp.zeros((LANES,), jnp.float32)
    for e in range(NUM_EXPERTS):
        acc_v[e, :] = zeros

    def body(g, _):
        off = pl.multiple_of(g * LANES, LANES)
        vs = [lt_v[e, pl.ds(off, LANES)] for e in range(NUM_EXPERTS)]

        m = vs[0]
        for e in range(1, NUM_EXPERTS):
            m = jnp.maximum(m, vs[e])

        i1 = jnp.full((LANES,), NUM_EXPERTS, jnp.int32)
        for e in range(NUM_EXPERTS - 1, -1, -1):
            i1 = jnp.where(vs[e] == m, jnp.full((LANES,), e, jnp.int32), i1)

        neg = jnp.full((LANES,), NEG_HUGE, jnp.float32)
        vm = [jnp.where(i1 == jnp.full((LANES,), e, jnp.int32), neg, vs[e])
              for e in range(NUM_EXPERTS)]
        l2 = vm[0]
        for e in range(1, NUM_EXPERTS):
            l2 = jnp.maximum(l2, vm[e])
        i2 = jnp.full((LANES,), NUM_EXPERTS, jnp.int32)
        for e in range(NUM_EXPERTS - 1, -1, -1):
            i2 = jnp.where(vm[e] == l2, jnp.full((LANES,), e, jnp.int32), i2)

        s = jnp.exp(vs[0] - m)
        for e in range(1, NUM_EXPERTS):
            s = s + jnp.exp(vs[e] - m)
        rs = 1.0 / s
        p1 = rs
        p2 = jnp.exp(l2 - m) * rs
        rden = 1.0 / (p1 + p2 + 1e-8)
        w1 = p1 * rden
        w2 = p2 * rden

        i1_v[pl.ds(off, LANES)] = i1
        i2_v[pl.ds(off, LANES)] = i2
        w1_v[pl.ds(off, LANES)] = w1
        w2_v[pl.ds(off, LANES)] = w2

        one = jnp.full((LANES,), 1.0, jnp.float32)
        for e in range(NUM_EXPERTS):
            ide = jnp.full((LANES,), e, jnp.int32)
            hit = jnp.where(i1 == ide, one, zeros) + \
                jnp.where(i2 == ide, one, zeros)
            acc_v[e, :] = acc_v[e, :] + hit
        return 0

    lax.fori_loop(0, groups, body, 0)

    pltpu.sync_copy(i1_v, oi_hbm.at[0, pl.ds(base, tok_per_w)])
    pltpu.sync_copy(i2_v, oi_hbm.at[1, pl.ds(base, tok_per_w)])
    pltpu.sync_copy(w1_v, ow_hbm.at[0, pl.ds(base, tok_per_w)])
    pltpu.sync_copy(w2_v, ow_hbm.at[1, pl.ds(base, tok_per_w)])
    pltpu.sync_copy(acc_v, oc_hbm.at[wid])


@jax.jit
def _router(gate_weight, hidden_flat):
    n_tokens = hidden_flat.shape[0]
    steps = CHUNK // TOKEN_BLOCK
    tok_per_w = CHUNK // NW
    mesh = plsc.VectorSubcoreMesh(core_axis_name="c", subcore_axis_name="s")

    its, wts, accs, lses, ents = [], [], [], [], []
    for c in range(N_CHUNKS):
        lt, lse_sum, ent_sum = pl.pallas_call(
            _gate_block,
            grid=(steps,),
            in_specs=[
                pl.BlockSpec((NUM_EXPERTS, D_MODEL), lambda i: (0, 0)),
                pl.BlockSpec((TOKEN_BLOCK, D_MODEL),
                             lambda i, c=c: (c * steps + i, 0)),
            ],
            out_specs=(
                pl.BlockSpec((NUM_EXPERTS, TOKEN_BLOCK), lambda i: (0, i)),
                pl.BlockSpec((1, 1), lambda i: (0, 0)),
                pl.BlockSpec((1, 1), lambda i: (0, 0)),
            ),
            out_shape=(
                jax.ShapeDtypeStruct((NUM_EXPERTS, CHUNK), jnp.float32),
                jax.ShapeDtypeStruct((1, 1), jnp.float32),
                jax.ShapeDtypeStruct((1, 1), jnp.float32),
            ),
            compiler_params=pltpu.CompilerParams(
                dimension_semantics=("arbitrary",),
            ),
        )(gate_weight, hidden_flat)

        route = functools.partial(
            pl.kernel, mesh=mesh,
            out_type=(
                jax.ShapeDtypeStruct((NUM_SELECTED, CHUNK), jnp.int32),
                jax.ShapeDtypeStruct((NUM_SELECTED, CHUNK), jnp.float32),
                jax.ShapeDtypeStruct((NW, NUM_EXPERTS, LANES), jnp.float32),
            ),
            scratch_types=[
                pltpu.VMEM((NUM_EXPERTS, tok_per_w), jnp.float32),
                pltpu.VMEM((tok_per_w,), jnp.int32),
                pltpu.VMEM((tok_per_w,), jnp.int32),
                pltpu.VMEM((tok_per_w,), jnp.float32),
                pltpu.VMEM((tok_per_w,), jnp.float32),
                pltpu.VMEM((NUM_EXPERTS, LANES), jnp.float32),
            ],
        )(functools.partial(_route_sc_body, tok_per_w=tok_per_w))
        it, wt, acc = route(lt)
        its.append(it)
        wts.append(wt)
        accs.append(acc)
        lses.append(lse_sum)
        ents.append(ent_sum)

    it = jnp.concatenate(its, axis=1)
    wt = jnp.concatenate(wts, axis=1)
    acc = sum(accs)
    lse_sum = sum(lses)
    ent_sum = sum(ents)
    return it, wt, acc, lse_sum, ent_sum


def kernel(hidden_states, gate_weight):
    batch_size, seq_len, d_model = hidden_states.shape
    num_tokens = batch_size * seq_len
    hidden_flat = hidden_states.reshape(num_tokens, d_model)

    it, wt, acc, lse_sum, ent_sum = _router(gate_weight, hidden_flat)

    expert_counts = jnp.sum(acc, axis=(0, 2))
    capacity = int(CAPACITY_FACTOR * num_tokens / NUM_EXPERTS * NUM_SELECTED)
    expert_overflow = jnp.sum(jnp.maximum(expert_counts - capacity, 0.0))
    capacity_overflow_pct = expert_overflow / num_tokens * 100.0
    z_loss = lse_sum[0, 0] / num_tokens * Z_LOSS_COEF
    gate_entropy = ent_sum[0, 0] / num_tokens
    expert_load_normalized = expert_counts / jnp.sum(expert_counts)
    ideal_load = 1.0 / NUM_EXPERTS
    expert_load_variance = jnp.mean((expert_load_normalized - ideal_load) ** 2)

    expert_indices = it.T.reshape(batch_size, seq_len, NUM_SELECTED)
    expert_weights = wt.T.reshape(batch_size, seq_len, NUM_SELECTED)
    routing_confidence = wt[0]
    return (expert_indices, expert_weights, expert_counts,
            capacity_overflow_pct, z_loss, gate_entropy,
            expert_load_variance, routing_confidence)


# speedup vs baseline: 1.0802x; 1.0365x over previous
import functools

import jax
import jax.numpy as jnp
from jax import lax
from jax.experimental import pallas as pl
from jax.experimental.pallas import tpu as pltpu
from jax.experimental.pallas import tpu_sc as plsc

D_MODEL = 2048
NUM_EXPERTS = 16
NUM_SELECTED = 2
CAPACITY_FACTOR = 1.25
Z_LOSS_COEF = 0.01

TOKEN_BLOCK = 1024
NEG_HUGE = -3.0e38

N_TOKENS = 16384
N_CHUNKS = 1
CHUNK = N_TOKENS // N_CHUNKS
NW = 32
LANES = 16


def _gate_block(w_ref, x0_ref, x1_ref, lt_ref, lse_ref, ent_ref):
    step = pl.program_id(0)

    l0 = lax.dot_general(
        w_ref[...], x0_ref[...],
        dimension_numbers=(((1,), (1,)), ((), ())),
        preferred_element_type=jnp.float32)
    l1 = lax.dot_general(
        w_ref[...], x1_ref[...],
        dimension_numbers=(((1,), (1,)), ((), ())),
        preferred_element_type=jnp.float32)
    logits = jnp.concatenate([l0, l1], axis=1)
    lt_ref[...] = logits

    m = jnp.max(logits, axis=0, keepdims=True)
    e = jnp.exp(logits - m)
    s = jnp.sum(e, axis=0, keepdims=True)
    lse = m + jnp.log(s)
    sel = jnp.sum(e * logits, axis=0, keepdims=True)
    ent = lse - sel / s
    block_lse = jnp.sum(lse)[None, None]
    block_ent = jnp.sum(ent)[None, None]

    @pl.when(step == 0)
    def _init():
        lse_ref[...] = block_lse
        ent_ref[...] = block_ent

    @pl.when(step != 0)
    def _acc():
        lse_ref[...] += block_lse
        ent_ref[...] += block_ent


def _route_sc_body(lt_hbm, oi_hbm, ow_hbm, oc_hbm,
                   lt_v, i1_v, i2_v, w1_v, w2_v, acc_v,
                   *, tok_per_w):
    groups = tok_per_w // LANES
    wid = lax.axis_index("s") * 2 + lax.axis_index("c")
    base = wid * tok_per_w
    pltpu.sync_copy(lt_hbm.at[:, pl.ds(base, tok_per_w)], lt_v)

    zeros = jnp.zeros((LANES,), jnp.float32)
    for e in range(NUM_EXPERTS):
        acc_v[e, :] = zeros

    def body(g, _):
        off = pl.multiple_of(g * LANES, LANES)
        vs = [lt_v[e, pl.ds(off, LANES)] for e in range(NUM_EXPERTS)]

        m = vs[0]
        for e in range(1, NUM_EXPERTS):
            m = jnp.maximum(m, vs[e])

        i1 = jnp.full((LANES,), NUM_EXPERTS, jnp.int32)
        for e in range(NUM_EXPERTS - 1, -1, -1):
            i1 = jnp.where(vs[e] == m, jnp.full((LANES,), e, jnp.int32), i1)

        neg = jnp.full((LANES,), NEG_HUGE, jnp.float32)
        vm = [jnp.where(i1 == jnp.full((LANES,), e, jnp.int32), neg, vs[e])
              for e in range(NUM_EXPERTS)]
        l2 = vm[0]
        for e in range(1, NUM_EXPERTS):
            l2 = jnp.maximum(l2, vm[e])
        i2 = jnp.full((LANES,), NUM_EXPERTS, jnp.int32)
        for e in range(NUM_EXPERTS - 1, -1, -1):
            i2 = jnp.where(vm[e] == l2, jnp.full((LANES,), e, jnp.int32), i2)

        s = jnp.exp(vs[0] - m)
        for e in range(1, NUM_EXPERTS):
            s = s + jnp.exp(vs[e] - m)
        rs = 1.0 / s
        p1 = rs
        p2 = jnp.exp(l2 - m) * rs
        rden = 1.0 / (p1 + p2 + 1e-8)
        w1 = p1 * rden
        w2 = p2 * rden

        i1_v[pl.ds(off, LANES)] = i1
        i2_v[pl.ds(off, LANES)] = i2
        w1_v[pl.ds(off, LANES)] = w1
        w2_v[pl.ds(off, LANES)] = w2

        one = jnp.full((LANES,), 1.0, jnp.float32)
        for e in range(NUM_EXPERTS):
            ide = jnp.full((LANES,), e, jnp.int32)
            hit = jnp.where(i1 == ide, one, zeros) + \
                jnp.where(i2 == ide, one, zeros)
            acc_v[e, :] = acc_v[e, :] + hit
        return 0

    lax.fori_loop(0, groups, body, 0)

    pltpu.sync_copy(i1_v, oi_hbm.at[0, pl.ds(base, tok_per_w)])
    pltpu.sync_copy(i2_v, oi_hbm.at[1, pl.ds(base, tok_per_w)])
    pltpu.sync_copy(w1_v, ow_hbm.at[0, pl.ds(base, tok_per_w)])
    pltpu.sync_copy(w2_v, ow_hbm.at[1, pl.ds(base, tok_per_w)])
    pltpu.sync_copy(acc_v, oc_hbm.at[wid])


@jax.jit
def _router(gate_weight, hidden_flat):
    n_tokens = hidden_flat.shape[0]
    steps = CHUNK // TOKEN_BLOCK
    tok_per_w = CHUNK // NW
    mesh = plsc.VectorSubcoreMesh(core_axis_name="c", subcore_axis_name="s")

    its, wts, accs, lses, ents = [], [], [], [], []
    for c in range(N_CHUNKS):
        lt, lse_sum, ent_sum = pl.pallas_call(
            _gate_block,
            grid=(steps,),
            in_specs=[
                pl.BlockSpec((NUM_EXPERTS, D_MODEL), lambda i: (0, 0)),
                pl.BlockSpec((TOKEN_BLOCK // 2, D_MODEL),
                             lambda i, c=c: (2 * (c * steps + i), 0)),
                pl.BlockSpec((TOKEN_BLOCK // 2, D_MODEL),
                             lambda i, c=c: (2 * (c * steps + i) + 1, 0)),
            ],
            out_specs=(
                pl.BlockSpec((NUM_EXPERTS, TOKEN_BLOCK), lambda i: (0, i)),
                pl.BlockSpec((1, 1), lambda i: (0, 0)),
                pl.BlockSpec((1, 1), lambda i: (0, 0)),
            ),
            out_shape=(
                jax.ShapeDtypeStruct((NUM_EXPERTS, CHUNK), jnp.float32),
                jax.ShapeDtypeStruct((1, 1), jnp.float32),
                jax.ShapeDtypeStruct((1, 1), jnp.float32),
            ),
            compiler_params=pltpu.CompilerParams(
                dimension_semantics=("arbitrary",),
            ),
        )(gate_weight, hidden_flat, hidden_flat)

        route = functools.partial(
            pl.kernel, mesh=mesh,
            out_type=(
                jax.ShapeDtypeStruct((NUM_SELECTED, CHUNK), jnp.int32),
                jax.ShapeDtypeStruct((NUM_SELECTED, CHUNK), jnp.float32),
                jax.ShapeDtypeStruct((NW, NUM_EXPERTS, LANES), jnp.float32),
            ),
            scratch_types=[
                pltpu.VMEM((NUM_EXPERTS, tok_per_w), jnp.float32),
                pltpu.VMEM((tok_per_w,), jnp.int32),
                pltpu.VMEM((tok_per_w,), jnp.int32),
                pltpu.VMEM((tok_per_w,), jnp.float32),
                pltpu.VMEM((tok_per_w,), jnp.float32),
                pltpu.VMEM((NUM_EXPERTS, LANES), jnp.float32),
            ],
        )(functools.partial(_route_sc_body, tok_per_w=tok_per_w))
        it, wt, acc = route(lt)
        its.append(it)
        wts.append(wt)
        accs.append(acc)
        lses.append(lse_sum)
        ents.append(ent_sum)

    it = jnp.concatenate(its, axis=1)
    wt = jnp.concatenate(wts, axis=1)
    acc = sum(accs)
    lse_sum = sum(lses)
    ent_sum = sum(ents)
    return it, wt, acc, lse_sum, ent_sum


def kernel(hidden_states, gate_weight):
    batch_size, seq_len, d_model = hidden_states.shape
    num_tokens = batch_size * seq_len
    hidden_flat = hidden_states.reshape(num_tokens, d_model)

    it, wt, acc, lse_sum, ent_sum = _router(gate_weight, hidden_flat)

    expert_counts = jnp.sum(acc, axis=(0, 2))
    capacity = int(CAPACITY_FACTOR * num_tokens / NUM_EXPERTS * NUM_SELECTED)
    expert_overflow = jnp.sum(jnp.maximum(expert_counts - capacity, 0.0))
    capacity_overflow_pct = expert_overflow / num_tokens * 100.0
    z_loss = lse_sum[0, 0] / num_tokens * Z_LOSS_COEF
    gate_entropy = ent_sum[0, 0] / num_tokens
    expert_load_normalized = expert_counts / jnp.sum(expert_counts)
    ideal_load = 1.0 / NUM_EXPERTS
    expert_load_variance = jnp.mean((expert_load_normalized - ideal_load) ** 2)

    expert_indices = it.T.reshape(batch_size, seq_len, NUM_SELECTED)
    expert_weights = wt.T.reshape(batch_size, seq_len, NUM_SELECTED)
    routing_confidence = wt[0]
    return (expert_indices, expert_weights, expert_counts,
            capacity_overflow_pct, z_loss, gate_entropy,
            expert_load_variance, routing_confidence)


# speedup vs baseline: 1.4910x; 1.3803x over previous
import jax
import jax.numpy as jnp
from jax import lax
from jax.experimental import pallas as pl
from jax.experimental.pallas import tpu as pltpu

D_MODEL = 2048
NUM_EXPERTS = 16
NUM_SELECTED = 2
CAPACITY_FACTOR = 1.25
Z_LOSS_COEF = 0.01

TOKEN_BLOCK = 1024
NEG_HUGE = -3.0e38


def _router_block(w_ref, x_ref, it_ref, wt_ref, cnt_ref, lse_ref, ent_ref):
    step = pl.program_id(0)

    logits = lax.dot_general(
        w_ref[...], x_ref[...],
        dimension_numbers=(((1,), (1,)), ((), ())),
        preferred_element_type=jnp.float32)

    m1 = jnp.max(logits, axis=0, keepdims=True)
    e = jnp.exp(logits - m1)
    s = jnp.sum(e, axis=0, keepdims=True)
    lse = m1 + jnp.log(s)
    sel = jnp.sum(e * logits, axis=0, keepdims=True)
    ent = lse - sel / s
    block_lse = jnp.sum(lse)[None, None]
    block_ent = jnp.sum(ent)[None, None]

    rows = lax.broadcasted_iota(jnp.int32, logits.shape, 0)
    big = jnp.int32(NUM_EXPERTS)
    i1 = jnp.min(jnp.where(logits == m1, rows, big), axis=0,
                 keepdims=True)
    vm = jnp.where(rows == i1, NEG_HUGE, logits)
    m2 = jnp.max(vm, axis=0, keepdims=True)
    i2 = jnp.min(jnp.where(vm == m2, rows, big), axis=0, keepdims=True)

    rs = 1.0 / s
    p1 = rs
    p2 = jnp.exp(m2 - m1) * rs
    rden = 1.0 / (p1 + p2 + 1e-8)
    w1 = p1 * rden
    w2 = p2 * rden

    it_ref[...] = jnp.concatenate([i1, i2], axis=0)
    wt_ref[...] = jnp.concatenate([w1, w2], axis=0)

    hit = (jnp.where(rows == i1, 1.0, 0.0) +
           jnp.where(rows == i2, 1.0, 0.0))
    block_cnt = jnp.sum(hit, axis=1, keepdims=True)

    @pl.when(step == 0)
    def _init():
        lse_ref[...] = block_lse
        ent_ref[...] = block_ent
        cnt_ref[...] = block_cnt

    @pl.when(step != 0)
    def _acc():
        lse_ref[...] += block_lse
        ent_ref[...] += block_ent
        cnt_ref[...] += block_cnt


@jax.jit
def _router(gate_weight, hidden_flat):
    n_tokens = hidden_flat.shape[0]
    grid = (n_tokens // TOKEN_BLOCK,)
    return pl.pallas_call(
        _router_block,
        grid=grid,
        in_specs=[
            pl.BlockSpec((NUM_EXPERTS, D_MODEL), lambda i: (0, 0)),
            pl.BlockSpec((TOKEN_BLOCK, D_MODEL), lambda i: (i, 0)),
        ],
        out_specs=(
            pl.BlockSpec((NUM_SELECTED, TOKEN_BLOCK), lambda i: (0, i)),
            pl.BlockSpec((NUM_SELECTED, TOKEN_BLOCK), lambda i: (0, i)),
            pl.BlockSpec((NUM_EXPERTS, 1), lambda i: (0, 0)),
            pl.BlockSpec((1, 1), lambda i: (0, 0)),
            pl.BlockSpec((1, 1), lambda i: (0, 0)),
        ),
        out_shape=(
            jax.ShapeDtypeStruct((NUM_SELECTED, n_tokens), jnp.int32),
            jax.ShapeDtypeStruct((NUM_SELECTED, n_tokens), jnp.float32),
            jax.ShapeDtypeStruct((NUM_EXPERTS, 1), jnp.float32),
            jax.ShapeDtypeStruct((1, 1), jnp.float32),
            jax.ShapeDtypeStruct((1, 1), jnp.float32),
        ),
        compiler_params=pltpu.CompilerParams(
            dimension_semantics=("arbitrary",),
        ),
    )(gate_weight, hidden_flat)


def kernel(hidden_states, gate_weight):
    batch_size, seq_len, d_model = hidden_states.shape
    num_tokens = batch_size * seq_len
    hidden_flat = hidden_states.reshape(num_tokens, d_model)

    it, wt, cnt, lse_sum, ent_sum = _router(gate_weight, hidden_flat)

    expert_counts = cnt[:, 0]
    capacity = int(CAPACITY_FACTOR * num_tokens / NUM_EXPERTS * NUM_SELECTED)
    expert_overflow = jnp.sum(jnp.maximum(expert_counts - capacity, 0.0))
    capacity_overflow_pct = expert_overflow / num_tokens * 100.0
    z_loss = lse_sum[0, 0] / num_tokens * Z_LOSS_COEF
    gate_entropy = ent_sum[0, 0] / num_tokens
    expert_load_normalized = expert_counts / jnp.sum(expert_counts)
    ideal_load = 1.0 / NUM_EXPERTS
    expert_load_variance = jnp.mean((expert_load_normalized - ideal_load) ** 2)

    expert_indices = it.T.reshape(batch_size, seq_len, NUM_SELECTED)
    expert_weights = wt.T.reshape(batch_size, seq_len, NUM_SELECTED)
    routing_confidence = wt[0]
    return (expert_indices, expert_weights, expert_counts,
            capacity_overflow_pct, z_loss, gate_entropy,
            expert_load_variance, routing_confidence)
